# baseline (speedup 1.0000x reference)
import jax
import jax.numpy as jnp
from jax.experimental import pallas as pl
from jax.experimental.pallas import tpu as pltpu

TILE2 = 512


def _router_kernel(xa, xb, xc, xd, w_ref, bias2_ref, out_ref):
    half = w_ref.shape[1] // 2

    def gate(lo_ref, hi_ref):
        return jax.lax.dot_general(
            lo_ref[...], w_ref[:, :half],
            dimension_numbers=(((1,), (1,)), ((), ())),
            preferred_element_type=jnp.float32,
        ) + jax.lax.dot_general(
            hi_ref[...], w_ref[:, half:],
            dimension_numbers=(((1,), (1,)), ((), ())),
            preferred_element_type=jnp.float32,
        )

    l_even = gate(xa, xb)
    l_odd = gate(xc, xd)
    l = jnp.concatenate([l_even, l_odd], axis=-1) + bias2_ref[...]
    m1 = jnp.max(l[:, :64], axis=-1, keepdims=True)
    m2 = jnp.max(l[:, 64:], axis=-1, keepdims=True)
    e1 = jnp.exp(l[:, :64] - m1)
    e2 = jnp.exp(l[:, 64:] - m2)
    p1 = e1 / jnp.sum(e1, axis=-1, keepdims=True)
    p2 = e2 / jnp.sum(e2, axis=-1, keepdims=True)
    out_ref[...] = jnp.concatenate([p1, p2], axis=-1)


@jax.jit
def kernel(x, W, bias):
    n_tokens, dim = x.shape
    n_experts = W.shape[0]
    x2 = x.reshape(n_tokens // 2, 2 * dim)
    q = (2 * dim) // 4
    grid = ((n_tokens // 2) // TILE2,)

    def mk(k):
        return pl.BlockSpec((TILE2, q), lambda i, k=k: (i, k))

    bias2 = jnp.concatenate([bias, bias]).reshape(1, 2 * n_experts)
    out = pl.pallas_call(
        _router_kernel,
        grid=grid,
        in_specs=[mk(k) for k in range(4)]
        + [
            pl.BlockSpec((n_experts, dim), lambda i: (0, 0)),
            pl.BlockSpec((1, 2 * n_experts), lambda i: (0, 0)),
        ],
        out_specs=pl.BlockSpec((TILE2, 2 * n_experts), lambda i: (i, 0)),
        out_shape=jax.ShapeDtypeStruct((n_tokens // 2, 2 * n_experts), jnp.float32),
        compiler_params=pltpu.CompilerParams(
            dimension_semantics=("arbitrary",),
        ),
    )(x2, x2, x2, x2, W, bias2)
    return out.reshape(n_tokens, n_experts)


# speedup vs baseline: 3.9694x; 3.9694x over previous
import jax
import jax.numpy as jnp
from jax.experimental import pallas as pl
from jax.experimental.pallas import tpu as pltpu

TILE = 1024
NSPLIT = 4


def _router_kernel(*refs):
    x_refs = refs[:NSPLIT]
    w_ref, bias_ref, out_ref = refs[NSPLIT:]
    q = x_refs[0].shape[1]
    logits = bias_ref[...]
    for k in range(NSPLIT):
        logits = logits + jax.lax.dot_general(
            x_refs[k][...], w_ref[:, k * q:(k + 1) * q],
            dimension_numbers=(((1,), (1,)), ((), ())),
            preferred_element_type=jnp.float32,
        )
    m = jnp.max(logits, axis=-1, keepdims=True)
    e = jnp.exp(logits - m)
    out_ref[...] = e / jnp.sum(e, axis=-1, keepdims=True)


@jax.jit
def kernel(x, W, bias):
    n_tokens, dim = x.shape
    n_experts = W.shape[0]
    q = dim // NSPLIT
    grid = (n_tokens // TILE,)

    def mk(k):
        return pl.BlockSpec((TILE, q), lambda i, k=k: (i, k))

    return pl.pallas_call(
        _router_kernel,
        grid=grid,
        in_specs=[mk(k) for k in range(NSPLIT)]
        + [
            pl.BlockSpec((n_experts, dim), lambda i: (0, 0)),
            pl.BlockSpec((1, n_experts), lambda i: (0, 0)),
        ],
        out_specs=pl.BlockSpec((TILE, n_experts), lambda i: (i, 0)),
        out_shape=jax.ShapeDtypeStruct((n_tokens, n_experts), jnp.float32),
        compiler_params=pltpu.CompilerParams(
            dimension_semantics=("arbitrary",),
        ),
    )(*([x] * NSPLIT), W, bias.reshape(1, n_experts))


# speedup vs baseline: 3.9754x; 1.0015x over previous
import jax
import jax.numpy as jnp
from jax.experimental import pallas as pl
from jax.experimental.pallas import tpu as pltpu

TILE = 1024
NSPLIT = 4


def _router_kernel(*refs):
    x_refs = refs[:NSPLIT]
    w_ref, bias_ref, out_ref = refs[NSPLIT:]
    q = x_refs[0].shape[1]
    logits = bias_ref[...]
    for k in range(NSPLIT):
        logits = logits + jax.lax.dot_general(
            x_refs[k][...], w_ref[:, k * q:(k + 1) * q],
            dimension_numbers=(((1,), (1,)), ((), ())),
            preferred_element_type=jnp.float32,
        )
    m = jnp.max(logits, axis=-1, keepdims=True)
    e = jnp.exp(logits - m)
    out_ref[...] = e / jnp.sum(e, axis=-1, keepdims=True)


@jax.jit
def kernel(x, W, bias):
    n_tokens, dim = x.shape
    n_experts = W.shape[0]
    q = dim // NSPLIT
    grid = (n_tokens // TILE,)

    def mk(k):
        return pl.BlockSpec((TILE, q), lambda i, k=k: (i, k))

    return pl.pallas_call(
        _router_kernel,
        grid=grid,
        in_specs=[mk(k) for k in range(NSPLIT)]
        + [
            pl.BlockSpec((n_experts, dim), lambda i: (0, 0)),
            pl.BlockSpec((1, n_experts), lambda i: (0, 0)),
        ],
        out_specs=pl.BlockSpec((TILE, n_experts), lambda i: (i, 0)),
        out_shape=jax.ShapeDtypeStruct((n_tokens, n_experts), jnp.float32),
        compiler_params=pltpu.CompilerParams(
            dimension_semantics=("arbitrary",),
            disable_bounds_checks=True,
            disable_semaphore_checks=True,
        ),
    )(*([x] * NSPLIT), W, bias.reshape(1, n_experts))
